# baseline (speedup 1.0000x reference)
import functools

import jax
import jax.numpy as jnp
from jax import lax
from jax.experimental import pallas as pl
from jax.experimental.pallas import tpu as pltpu
from jax.experimental.pallas import tpu_sc as plsc

N_NODES = 10000
N_EDGES = 320000
D_FEAT = 128
N_CHANNELS = 128

NC = 2
NS = 16
L = 16

CH = N_CHANNELS // NC
EDGE_CHUNK = 80
CHUNK_ROWS = N_EDGES // EDGE_CHUNK
TILE_CHUNKS = CHUNK_ROWS // NS
NBUF = 5
N_STAGES = 2
STAGE = TILE_CHUNKS // N_STAGES
ROWS_PER_TILE = N_NODES // NS
SLAB = 125
N_SLABS = ROWS_PER_TILE // SLAB

SELU_SCALE = 1.0507009873554805
SELU_ALPHA = 1.6732632423543772


def _matmul_body(f_ref, w_ref, b_ref, o_ref):
    o_ref[...] = (
        jnp.dot(f_ref[...], w_ref[...], preferred_element_type=jnp.float32)
        + b_ref[...]
    )


def _transform(features, weight, bias):
    blk = 1000
    grid = (N_NODES // blk,)
    return pl.pallas_call(
        _matmul_body,
        grid=grid,
        in_specs=[
            pl.BlockSpec((blk, D_FEAT), lambda i: (i, 0)),
            pl.BlockSpec((D_FEAT, N_CHANNELS), lambda i: (0, 0)),
            pl.BlockSpec((1, N_CHANNELS), lambda i: (0, 0)),
        ],
        out_specs=pl.BlockSpec((blk, N_CHANNELS), lambda i: (i, 0)),
        out_shape=jax.ShapeDtypeStruct((N_NODES, N_CHANNELS), jnp.float32),
    )(features, weight, bias.reshape(1, N_CHANNELS))


def _sc_body(table_hbm, src_hbm, dst_hbm, ew_hbm, skip_hbm, out_hbm,
             src_v, dst_v, ew_v, rows_g, rows_s, buf_v, skip_v, acc_sh,
             sems, ssems):
    c = lax.axis_index("c")
    s = lax.axis_index("s")
    zeros = jnp.zeros((L,), jnp.float32)

    @plsc.parallel_loop(0, SLAB)
    def _zrow(r):
        for t in range(CH // L):
            buf_v[r, pl.ds(t * L, L)] = zeros
    for q in range(N_SLABS):
        pltpu.sync_copy(buf_v, acc_sh.at[pl.ds(s * ROWS_PER_TILE + q * SLAB, SLAB)])

    plsc.subcore_barrier()

    def _scale(j, r):
        jfull = jnp.full((L,), j, jnp.int32)

        @plsc.parallel_loop(0, EDGE_CHUNK, unroll=8)
        def _edge(k):
            w = plsc.load_gather(ew_v, [jfull, jnp.full((L,), k, jnp.int32)])
            for t in range(CH // L):
                sl = pl.ds(t * L, L)
                rows_s[r][k, sl] = rows_g[r][k, sl] * w

    DEPTH = NBUF - 1
    n_blk = STAGE // NBUF
    for h in range(N_STAGES):
        base = s * TILE_CHUNKS + h * STAGE
        pltpu.sync_copy(src_hbm.at[pl.ds(base, STAGE)], src_v)
        pltpu.sync_copy(dst_hbm.at[pl.ds(base, STAGE)], dst_v)
        pltpu.sync_copy(ew_hbm.at[pl.ds(base, STAGE)], ew_v)

        @plsc.parallel_loop(0, STAGE)
        def _fix(r):
            for t in range(EDGE_CHUNK // L):
                sl = pl.ds(t * L, L)
                src_v[r, sl] = src_v[r, sl] * 2 + c

        for j in range(DEPTH):
            pltpu.async_copy(table_hbm.at[src_v.at[j]], rows_g[j], sems[j])

        def _block(i, _):
            for b in range(NBUF):
                j = i * NBUF + b
                pltpu.make_async_copy(
                    table_hbm.at[src_v.at[j]], rows_g[b], sems[b]).wait()

                @pl.when(i > 0)
                def _():
                    pltpu.make_async_copy(
                        rows_s[b], acc_sh.at[dst_v.at[j]], ssems[b]).wait()

                _scale(j, b)
                pltpu.async_copy(
                    rows_s[b], acc_sh.at[dst_v.at[j]], ssems[b], add=True)
                jn = j + DEPTH
                bn = (b + DEPTH) % NBUF
                if b == 0:
                    pltpu.async_copy(
                        table_hbm.at[src_v.at[jn]], rows_g[bn], sems[bn])
                else:
                    @pl.when(i < n_blk - 1)
                    def _():
                        pltpu.async_copy(
                            table_hbm.at[src_v.at[jn]], rows_g[bn], sems[bn])
            return _
        lax.fori_loop(0, n_blk, _block, None)

        for b in range(NBUF):
            pltpu.make_async_copy(
                rows_s[b], acc_sh.at[dst_v.at[0]], ssems[b]).wait()

    plsc.subcore_barrier()

    pltpu.sync_copy(skip_hbm.at[pl.ds(c * CH, CH)], skip_v)
    for q in range(N_SLABS):
        row0 = s * ROWS_PER_TILE + q * SLAB
        pltpu.sync_copy(acc_sh.at[pl.ds(row0, SLAB)], buf_v)

        @plsc.parallel_loop(0, SLAB)
        def _selu_row(r):
            for t in range(CH // L):
                sl = pl.ds(t * L, L)
                x = buf_v[r, sl] + skip_v[sl]
                y = SELU_SCALE * jnp.where(
                    x > 0.0, x, SELU_ALPHA * (jnp.exp(x) - 1.0))
                buf_v[r, sl] = y

        pltpu.sync_copy(buf_v, out_hbm.at[pl.ds(row0, SLAB), pl.ds(c * CH, CH)])


@functools.partial(
    pl.kernel,
    mesh=plsc.VectorSubcoreMesh(core_axis_name="c", subcore_axis_name="s"),
    out_type=jax.ShapeDtypeStruct((N_NODES, N_CHANNELS), jnp.float32),
    compiler_params=pltpu.CompilerParams(
        use_tc_tiling_on_sc=False, needs_layout_passes=False),
    scratch_types=[
        pltpu.VMEM((STAGE, EDGE_CHUNK), jnp.int32),
        pltpu.VMEM((STAGE, EDGE_CHUNK), jnp.int32),
        pltpu.VMEM((STAGE, EDGE_CHUNK), jnp.float32),
        *[pltpu.VMEM((EDGE_CHUNK, CH), jnp.float32) for _ in range(2 * NBUF)],
        pltpu.VMEM((SLAB, CH), jnp.float32),
        pltpu.VMEM((CH,), jnp.float32),
        pltpu.VMEM_SHARED((N_NODES, CH), jnp.float32),
        *[pltpu.SemaphoreType.DMA for _ in range(2 * NBUF)],
    ],
)
def _sc_aggregate(table_hbm, src_hbm, dst_hbm, ew_hbm, skip_hbm, out_hbm,
                  src_v, dst_v, ew_v, g0, g1, g2, g3, g4, r0, r1, r2, r3, r4,
                  buf_v, skip_v, acc_sh, s0, s1, s2, s3, s4,
                  t0, t1, t2, t3, t4):
    _sc_body(table_hbm, src_hbm, dst_hbm, ew_hbm, skip_hbm, out_hbm,
             src_v, dst_v, ew_v, [g0, g1, g2, g3, g4], [r0, r1, r2, r3, r4],
             buf_v, skip_v, acc_sh,
             [s0, s1, s2, s3, s4], [t0, t1, t2, t3, t4])


def kernel(features, edge_index, edge_weight, kernel, bias, skip_weight):
    out = _transform(features, kernel, bias)
    table = out.reshape(2 * N_NODES, CH)
    src = edge_index[0].astype(jnp.int32).reshape(CHUNK_ROWS, EDGE_CHUNK)
    dst = edge_index[1].astype(jnp.int32).reshape(CHUNK_ROWS, EDGE_CHUNK)
    ew = edge_weight.reshape(CHUNK_ROWS, EDGE_CHUNK)
    return _sc_aggregate(table, src, dst, ew, skip_weight)


# speedup vs baseline: 1.1112x; 1.1112x over previous
import functools

import jax
import jax.numpy as jnp
import numpy as np
from jax import lax
from jax.experimental import pallas as pl
from jax.experimental.pallas import tpu as pltpu
from jax.experimental.pallas import tpu_sc as plsc

N_NODES = 10000
N_EDGES = 320000
D_FEAT = 128
N_CHANNELS = 128

NC = 2
NS = 16
L = 16

CH = N_CHANNELS // NC
EDGE_CHUNK = 80
CHUNK_ROWS = N_EDGES // EDGE_CHUNK
TILE_CHUNKS = CHUNK_ROWS // NS
NBUF = 5
N_STAGES = 2
STAGE = TILE_CHUNKS // N_STAGES
ROWS_PER_TILE = N_NODES // NS
SLAB = 125
N_SLABS = ROWS_PER_TILE // SLAB

SELU_SCALE = 1.0507009873554805
SELU_ALPHA = 1.6732632423543772

_PACK_PERM = np.empty((N_CHANNELS,), np.int64)
for _base in range(0, N_CHANNELS, 2 * L):
    for _i in range(L):
        _PACK_PERM[_base + 2 * _i] = _base + _i
        _PACK_PERM[_base + 2 * _i + 1] = _base + L + _i


def _matmul_body(f_ref, w_ref, b_ref, o_ref):
    o_ref[...] = (
        jnp.dot(f_ref[...], w_ref[...], preferred_element_type=jnp.float32)
        + b_ref[...]
    ).astype(jnp.bfloat16)


def _transform(features, weight, bias):
    blk = 1000
    grid = (N_NODES // blk,)
    return pl.pallas_call(
        _matmul_body,
        grid=grid,
        in_specs=[
            pl.BlockSpec((blk, D_FEAT), lambda i: (i, 0)),
            pl.BlockSpec((D_FEAT, N_CHANNELS), lambda i: (0, 0)),
            pl.BlockSpec((1, N_CHANNELS), lambda i: (0, 0)),
        ],
        out_specs=pl.BlockSpec((blk, N_CHANNELS), lambda i: (i, 0)),
        out_shape=jax.ShapeDtypeStruct((N_NODES, N_CHANNELS), jnp.bfloat16),
    )(features, weight, bias.reshape(1, N_CHANNELS))


def _sc_body(table_hbm, src_hbm, dst_hbm, ew_hbm, skip_hbm, out_hbm,
             src_v, dst_v, ew_v, rows_g, rows_s, buf_v, skip_v, acc_sh,
             sems, ssems):
    c = lax.axis_index("c")
    s = lax.axis_index("s")
    zeros = jnp.zeros((L,), jnp.float32)

    @plsc.parallel_loop(0, SLAB)
    def _zrow(r):
        for t in range(CH // L):
            buf_v[r, pl.ds(t * L, L)] = zeros
    for q in range(N_SLABS):
        pltpu.sync_copy(buf_v, acc_sh.at[pl.ds(s * ROWS_PER_TILE + q * SLAB, SLAB)])

    plsc.subcore_barrier()

    def _scale(j, r):
        jfull = jnp.full((L,), j, jnp.int32)

        @plsc.parallel_loop(0, EDGE_CHUNK, unroll=4)
        def _edge(k):
            w = plsc.load_gather(ew_v, [jfull, jnp.full((L,), k, jnp.int32)])
            for t in range(CH // (2 * L)):
                ab = rows_g[r][k, pl.ds(t * 2 * L, 2 * L)]
                a, b = plsc.unpack(ab, format=plsc.PackFormat.INTERLEAVED)
                rows_s[r][k, pl.ds(t * 2 * L, L)] = a * w
                rows_s[r][k, pl.ds(t * 2 * L + L, L)] = b * w

    DEPTH = NBUF - 1
    n_blk = STAGE // NBUF
    for h in range(N_STAGES):
        base = s * TILE_CHUNKS + h * STAGE
        pltpu.sync_copy(src_hbm.at[pl.ds(base, STAGE)], src_v)
        pltpu.sync_copy(dst_hbm.at[pl.ds(base, STAGE)], dst_v)
        pltpu.sync_copy(ew_hbm.at[pl.ds(base, STAGE)], ew_v)

        @plsc.parallel_loop(0, STAGE)
        def _fix(r):
            for t in range(EDGE_CHUNK // L):
                sl = pl.ds(t * L, L)
                src_v[r, sl] = src_v[r, sl] * 2 + c

        for j in range(DEPTH):
            pltpu.async_copy(table_hbm.at[src_v.at[j]], rows_g[j], sems[j])

        def _block(i, _):
            for b in range(NBUF):
                j = i * NBUF + b
                pltpu.make_async_copy(
                    table_hbm.at[src_v.at[j]], rows_g[b], sems[b]).wait()

                @pl.when(i > 0)
                def _():
                    pltpu.make_async_copy(
                        rows_s[b], acc_sh.at[dst_v.at[j]], ssems[b]).wait()

                _scale(j, b)
                pltpu.async_copy(
                    rows_s[b], acc_sh.at[dst_v.at[j]], ssems[b], add=True)
                jn = j + DEPTH
                bn = (b + DEPTH) % NBUF
                if b == 0:
                    pltpu.async_copy(
                        table_hbm.at[src_v.at[jn]], rows_g[bn], sems[bn])
                else:
                    @pl.when(i < n_blk - 1)
                    def _():
                        pltpu.async_copy(
                            table_hbm.at[src_v.at[jn]], rows_g[bn], sems[bn])
            return _
        lax.fori_loop(0, n_blk, _block, None)

        for b in range(NBUF):
            pltpu.make_async_copy(
                rows_s[b], acc_sh.at[dst_v.at[0]], ssems[b]).wait()

    plsc.subcore_barrier()

    pltpu.sync_copy(skip_hbm.at[pl.ds(c * CH, CH)], skip_v)
    for q in range(N_SLABS):
        row0 = s * ROWS_PER_TILE + q * SLAB
        pltpu.sync_copy(acc_sh.at[pl.ds(row0, SLAB)], buf_v)

        @plsc.parallel_loop(0, SLAB)
        def _selu_row(r):
            for t in range(CH // L):
                sl = pl.ds(t * L, L)
                x = buf_v[r, sl] + skip_v[sl]
                y = SELU_SCALE * jnp.where(
                    x > 0.0, x, SELU_ALPHA * (jnp.exp(x) - 1.0))
                buf_v[r, sl] = y

        pltpu.sync_copy(buf_v, out_hbm.at[pl.ds(row0, SLAB), pl.ds(c * CH, CH)])


@functools.partial(
    pl.kernel,
    mesh=plsc.VectorSubcoreMesh(core_axis_name="c", subcore_axis_name="s"),
    out_type=jax.ShapeDtypeStruct((N_NODES, N_CHANNELS), jnp.float32),
    compiler_params=pltpu.CompilerParams(
        use_tc_tiling_on_sc=False, needs_layout_passes=False),
    scratch_types=[
        pltpu.VMEM((STAGE, EDGE_CHUNK), jnp.int32),
        pltpu.VMEM((STAGE, EDGE_CHUNK), jnp.int32),
        pltpu.VMEM((STAGE, EDGE_CHUNK), jnp.float32),
        *[pltpu.VMEM((EDGE_CHUNK, CH), jnp.bfloat16) for _ in range(NBUF)],
        *[pltpu.VMEM((EDGE_CHUNK, CH), jnp.float32) for _ in range(NBUF)],
        pltpu.VMEM((SLAB, CH), jnp.float32),
        pltpu.VMEM((CH,), jnp.float32),
        pltpu.VMEM_SHARED((N_NODES, CH), jnp.float32),
        *[pltpu.SemaphoreType.DMA for _ in range(2 * NBUF)],
    ],
)
def _sc_aggregate(table_hbm, src_hbm, dst_hbm, ew_hbm, skip_hbm, out_hbm,
                  src_v, dst_v, ew_v, g0, g1, g2, g3, g4, r0, r1, r2, r3, r4,
                  buf_v, skip_v, acc_sh, s0, s1, s2, s3, s4,
                  t0, t1, t2, t3, t4):
    _sc_body(table_hbm, src_hbm, dst_hbm, ew_hbm, skip_hbm, out_hbm,
             src_v, dst_v, ew_v, [g0, g1, g2, g3, g4], [r0, r1, r2, r3, r4],
             buf_v, skip_v, acc_sh,
             [s0, s1, s2, s3, s4], [t0, t1, t2, t3, t4])


def kernel(features, edge_index, edge_weight, kernel, bias, skip_weight):
    perm = jnp.asarray(_PACK_PERM)
    out = _transform(features, kernel[:, perm], bias[perm])
    table = out.reshape(2 * N_NODES, CH)
    src = edge_index[0].astype(jnp.int32).reshape(CHUNK_ROWS, EDGE_CHUNK)
    dst = edge_index[1].astype(jnp.int32).reshape(CHUNK_ROWS, EDGE_CHUNK)
    ew = edge_weight.reshape(CHUNK_ROWS, EDGE_CHUNK)
    return _sc_aggregate(table, src, dst, ew, skip_weight)


# speedup vs baseline: 1.3671x; 1.2303x over previous
import functools

import jax
import jax.numpy as jnp
import numpy as np
from jax import lax
from jax.experimental import pallas as pl
from jax.experimental.pallas import tpu as pltpu
from jax.experimental.pallas import tpu_sc as plsc

N_NODES = 10000
N_EDGES = 320000
D_FEAT = 128
N_CHANNELS = 128

NC = 2
NS = 16
L = 16

CH = N_CHANNELS // NC
EDGE_CHUNK = 80
CHUNK_ROWS = N_EDGES // EDGE_CHUNK
TILE_CHUNKS = CHUNK_ROWS // NS
NBUF = 5
N_STAGES = 1
STAGE = TILE_CHUNKS // N_STAGES
ROWS_PER_TILE = N_NODES // NS
SLAB = 125
N_SLABS = ROWS_PER_TILE // SLAB

SELU_SCALE = 1.0507009873554805
SELU_ALPHA = 1.6732632423543772

_PACK_PERM = np.empty((N_CHANNELS,), np.int64)
for _base in range(0, N_CHANNELS, 2 * L):
    for _i in range(L):
        _PACK_PERM[_base + 2 * _i] = _base + _i
        _PACK_PERM[_base + 2 * _i + 1] = _base + L + _i


def _matmul_body(f_ref, w_ref, b_ref, o_ref):
    o_ref[...] = (
        jnp.dot(f_ref[...], w_ref[...], preferred_element_type=jnp.float32)
        + b_ref[...]
    ).astype(jnp.bfloat16)


def _transform(features, weight, bias):
    blk = 1000
    grid = (N_NODES // blk,)
    return pl.pallas_call(
        _matmul_body,
        grid=grid,
        in_specs=[
            pl.BlockSpec((blk, D_FEAT), lambda i: (i, 0)),
            pl.BlockSpec((D_FEAT, N_CHANNELS), lambda i: (0, 0)),
            pl.BlockSpec((1, N_CHANNELS), lambda i: (0, 0)),
        ],
        out_specs=pl.BlockSpec((blk, N_CHANNELS), lambda i: (i, 0)),
        out_shape=jax.ShapeDtypeStruct((N_NODES, N_CHANNELS), jnp.bfloat16),
    )(features, weight, bias.reshape(1, N_CHANNELS))


def _sc_body(table_hbm, src_hbm, dst_hbm, ew_hbm, skip_hbm, out_hbm,
             src_v, dst_v, ew_v, rows_g, rows_s, buf_v, bufz_v, skip_v,
             acc_sh, sems, ssems):
    c = lax.axis_index("c")
    s = lax.axis_index("s")
    zeros_bf = jnp.zeros((2 * L,), jnp.bfloat16)

    @plsc.parallel_loop(0, SLAB)
    def _zrow(r):
        for t in range(CH // (2 * L)):
            bufz_v[r, pl.ds(t * 2 * L, 2 * L)] = zeros_bf
    for q in range(N_SLABS):
        pltpu.sync_copy(bufz_v, acc_sh.at[pl.ds(s * ROWS_PER_TILE + q * SLAB, SLAB)])

    plsc.subcore_barrier()

    def _scale(j, r):
        jfull = jnp.full((L,), j, jnp.int32)

        @plsc.parallel_loop(0, EDGE_CHUNK, unroll=4)
        def _edge(k):
            w = plsc.load_gather(ew_v, [jfull, jnp.full((L,), k, jnp.int32)])
            wbf = plsc.pack(w, w, format=plsc.PackFormat.INTERLEAVED)
            for t in range(CH // (2 * L)):
                sl = pl.ds(t * 2 * L, 2 * L)
                rows_s[r][k, sl] = rows_g[r][k, sl] * wbf

    DEPTH = NBUF - 1
    n_blk = STAGE // NBUF
    for h in range(N_STAGES):
        base = s * TILE_CHUNKS + h * STAGE
        pltpu.sync_copy(src_hbm.at[pl.ds(base, STAGE)], src_v)
        pltpu.sync_copy(dst_hbm.at[pl.ds(base, STAGE)], dst_v)
        pltpu.sync_copy(ew_hbm.at[pl.ds(base, STAGE)], ew_v)

        @plsc.parallel_loop(0, STAGE)
        def _fix(r):
            for t in range(EDGE_CHUNK // L):
                sl = pl.ds(t * L, L)
                src_v[r, sl] = src_v[r, sl] * 2 + c

        for j in range(DEPTH):
            pltpu.async_copy(table_hbm.at[src_v.at[j]], rows_g[j], sems[j])

        def _block(i, _):
            for b in range(NBUF):
                j = i * NBUF + b
                pltpu.make_async_copy(
                    table_hbm.at[src_v.at[j]], rows_g[b], sems[b]).wait()

                @pl.when(i > 0)
                def _():
                    pltpu.make_async_copy(
                        rows_s[b], acc_sh.at[dst_v.at[j]], ssems[b]).wait()

                _scale(j, b)
                pltpu.async_copy(
                    rows_s[b], acc_sh.at[dst_v.at[j]], ssems[b], add=True)
                jn = j + DEPTH
                bn = (b + DEPTH) % NBUF
                if b == 0:
                    pltpu.async_copy(
                        table_hbm.at[src_v.at[jn]], rows_g[bn], sems[bn])
                else:
                    @pl.when(i < n_blk - 1)
                    def _():
                        pltpu.async_copy(
                            table_hbm.at[src_v.at[jn]], rows_g[bn], sems[bn])
            return _
        lax.fori_loop(0, n_blk, _block, None)

        for b in range(NBUF):
            pltpu.make_async_copy(
                rows_s[b], acc_sh.at[dst_v.at[0]], ssems[b]).wait()

    plsc.subcore_barrier()

    pltpu.sync_copy(skip_hbm.at[pl.ds(c * CH, CH)], skip_v)
    for q in range(N_SLABS):
        row0 = s * ROWS_PER_TILE + q * SLAB
        pltpu.sync_copy(acc_sh.at[pl.ds(row0, SLAB)], bufz_v)

        @plsc.parallel_loop(0, SLAB)
        def _selu_row(r):
            for t in range(CH // (2 * L)):
                ab = bufz_v[r, pl.ds(t * 2 * L, 2 * L)]
                a, b = plsc.unpack(ab, format=plsc.PackFormat.INTERLEAVED)
                for u, half in ((0, a), (1, b)):
                    sl = pl.ds(t * 2 * L + u * L, L)
                    x = half + skip_v[sl]
                    y = SELU_SCALE * jnp.where(
                        x > 0.0, x, SELU_ALPHA * (jnp.exp(x) - 1.0))
                    buf_v[r, sl] = y

        pltpu.sync_copy(buf_v, out_hbm.at[pl.ds(row0, SLAB), pl.ds(c * CH, CH)])


@functools.partial(
    pl.kernel,
    mesh=plsc.VectorSubcoreMesh(core_axis_name="c", subcore_axis_name="s"),
    out_type=jax.ShapeDtypeStruct((N_NODES, N_CHANNELS), jnp.float32),
    compiler_params=pltpu.CompilerParams(
        use_tc_tiling_on_sc=False, needs_layout_passes=False),
    scratch_types=[
        pltpu.VMEM((STAGE, EDGE_CHUNK), jnp.int32),
        pltpu.VMEM((STAGE, EDGE_CHUNK), jnp.int32),
        pltpu.VMEM((STAGE, EDGE_CHUNK), jnp.float32),
        *[pltpu.VMEM((EDGE_CHUNK, CH), jnp.bfloat16) for _ in range(2 * NBUF)],
        pltpu.VMEM((SLAB, CH), jnp.float32),
        pltpu.VMEM((SLAB, CH), jnp.bfloat16),
        pltpu.VMEM((CH,), jnp.float32),
        pltpu.VMEM_SHARED((N_NODES, CH), jnp.bfloat16),
        *[pltpu.SemaphoreType.DMA for _ in range(2 * NBUF)],
    ],
)
def _sc_aggregate(table_hbm, src_hbm, dst_hbm, ew_hbm, skip_hbm, out_hbm,
                  src_v, dst_v, ew_v, g0, g1, g2, g3, g4, r0, r1, r2, r3, r4,
                  buf_v, bufz_v, skip_v, acc_sh, s0, s1, s2, s3, s4,
                  t0, t1, t2, t3, t4):
    _sc_body(table_hbm, src_hbm, dst_hbm, ew_hbm, skip_hbm, out_hbm,
             src_v, dst_v, ew_v, [g0, g1, g2, g3, g4], [r0, r1, r2, r3, r4],
             buf_v, bufz_v, skip_v, acc_sh,
             [s0, s1, s2, s3, s4], [t0, t1, t2, t3, t4])


def kernel(features, edge_index, edge_weight, kernel, bias, skip_weight):
    perm = jnp.asarray(_PACK_PERM)
    out = _transform(features, kernel[:, perm], bias[perm])
    table = out.reshape(2 * N_NODES, CH)
    src = edge_index[0].astype(jnp.int32).reshape(CHUNK_ROWS, EDGE_CHUNK)
    dst = edge_index[1].astype(jnp.int32).reshape(CHUNK_ROWS, EDGE_CHUNK)
    ew = edge_weight.reshape(CHUNK_ROWS, EDGE_CHUNK)
    return _sc_aggregate(table, src, dst, ew, skip_weight)
